# Initial kernel scaffold; baseline (speedup 1.0000x reference)
import jax
import jax.numpy as jnp
from jax.experimental import pallas as pl


def kernel(pred, k):
    raise NotImplementedError("write your pallas kernel here")


# speedup vs baseline: 28.6000x; 28.6000x over previous
import functools

import jax
import jax.numpy as jnp
from jax import lax
from jax.experimental import pallas as pl
from jax.experimental.pallas import tpu as pltpu

_KEY_NEG_INF = -2139095041
_KEY_POS_INF = 2139095040


def _unkey(v):
    b = v ^ ((v >> 31) & jnp.int32(0x7FFFFFFF))
    return lax.bitcast_convert_type(b, jnp.float32)


def _threshold_body(r_ref, pred_ref, thr_ref, *, chunk):
    B, S = pred_ref.shape
    rv = jnp.full((B, 1), r_ref[0], jnp.int32)

    def count_lt(fmid):
        def body(c, acc):
            idx = pl.multiple_of(c * chunk, chunk)
            x = pred_ref[:, pl.ds(idx, chunk)]
            return acc + jnp.where(x < fmid, 1, 0).astype(jnp.int32)

        acc = lax.fori_loop(0, S // chunk, body,
                            jnp.zeros((B, chunk), jnp.int32))
        return jnp.sum(acc, axis=1, keepdims=True)

    n_neg = count_lt(jnp.zeros((B, 1), jnp.float32))
    is_neg = n_neg > rv
    lo = jnp.where(is_neg, jnp.int32(_KEY_NEG_INF), jnp.int32(0))
    hi = jnp.where(is_neg, jnp.int32(-1), jnp.int32(_KEY_POS_INF))

    def step(_, lohi):
        lo, hi = lohi
        mid = lo + lax.shift_right_logical(hi - lo + 1, 1)
        take = count_lt(_unkey(mid)) <= rv
        return jnp.where(take, mid, lo), jnp.where(take, hi, mid - 1)

    lo, hi = lax.fori_loop(0, 31, step, (lo, hi))
    thr_ref[...] = jnp.broadcast_to(_unkey(lo), (B, 128))


def _mask_body(pred_ref, thr_ref, keep_ref, pruned_ref):
    x = pred_ref[...]
    m = x > thr_ref[:, 0:1]
    keep_ref[...] = m
    pruned_ref[...] = jnp.where(m, x, jnp.float32(0.0))


@functools.partial(jax.jit, static_argnames=("interpret",))
def _run(pred, k, interpret=False):
    B, S = pred.shape
    r = (jnp.int32(S - 1) - k.astype(jnp.int32)).reshape((1,))
    chunk = min(S, 1024)

    thr = pl.pallas_call(
        functools.partial(_threshold_body, chunk=chunk),
        out_shape=jax.ShapeDtypeStruct((B, 128), jnp.float32),
        in_specs=[
            pl.BlockSpec(memory_space=pltpu.SMEM),
            pl.BlockSpec(memory_space=pltpu.VMEM),
        ],
        compiler_params=pltpu.CompilerParams(
            vmem_limit_bytes=100 * 1024 * 1024),
        interpret=interpret,
    )(r, pred)

    blk = min(S, 16384)
    keep, pruned = pl.pallas_call(
        _mask_body,
        grid=(S // blk,),
        in_specs=[
            pl.BlockSpec((B, blk), lambda j: (0, j)),
            pl.BlockSpec((B, 128), lambda j: (0, 0)),
        ],
        out_specs=[
            pl.BlockSpec((B, blk), lambda j: (0, j)),
            pl.BlockSpec((B, blk), lambda j: (0, j)),
        ],
        out_shape=[
            jax.ShapeDtypeStruct((B, S), jnp.bool_),
            jax.ShapeDtypeStruct((B, S), jnp.float32),
        ],
        interpret=interpret,
    )(pred, thr)
    return keep, pruned


def kernel(pred, k):
    return _run(pred, jnp.asarray(k))
